# baseline (speedup 1.0000x reference)
import jax
import jax.numpy as jnp
from jax.experimental import pallas as pl

B, G, T, N = 4, 8, 32, 100
D = 256
R = 4
MD = 32
TN = T * N
BG = B * G
ROW_CHUNKS = 4


def _shift_rows(x, s):
    return jnp.concatenate([jnp.zeros((s,) + x.shape[1:], x.dtype), x[:-s]], axis=0)


def _roll_rows(x, s):
    return jnp.concatenate([x[-s:], x[:-s]], axis=0)


def _eff(vals, hasf):
    V = vals * hasf
    H = hasf
    s = 1
    while s < T:
        V = jnp.where(H > 0.5, V, _shift_rows(V, s * N))
        H = jnp.maximum(H, _shift_rows(H, s * N))
        s *= 2
    return _shift_rows(V, N), _shift_rows(H, N)


def _gelu(x):
    return x * 0.5 * (1.0 + jax.lax.erf(x * 0.7071067811865476))


def _ffn(x32, w1_ref, b1_ref, w2_ref, b2_ref, chunks=4):
    bf = jnp.bfloat16
    xb = x32.astype(bf)
    hdim = w1_ref.shape[1]
    cs = hdim // chunks
    b1 = b1_ref[...]
    y = jnp.broadcast_to(b2_ref[...], (x32.shape[0], w2_ref.shape[1]))
    for c in range(chunks):
        h = (jnp.dot(xb, w1_ref[:, c * cs:(c + 1) * cs],
                     preferred_element_type=jnp.float32)
             + b1[c * cs:(c + 1) * cs]).astype(bf)
        y = y + jnp.dot(_gelu(h), w2_ref[c * cs:(c + 1) * cs, :],
                        preferred_element_type=jnp.float32)
    return y


def _layernorm(x, g, b):
    m = jnp.mean(x, axis=-1, keepdims=True)
    v = jnp.mean((x - m) ** 2, axis=-1, keepdims=True)
    return (x - m) * jax.lax.rsqrt(v + 1e-5) * g + b


def _fused_kernel(feat_ref, boxes_ref, masks_ref,
                  wa1_ref, ba1_ref, wa2_ref, ba2_ref, g1_ref, be1_ref,
                  we1_ref, bee1_ref, we2_ref, bee2_ref,
                  wf1a_ref, wf1b_ref, bf1_ref, wf2_ref, bf2_ref,
                  wn1_ref, bn1_ref, wn2_ref, bn2_ref, g2_ref, be2_ref,
                  f_out_ref, mot_out_ref):
    bf = jnp.bfloat16

    bo = boxes_ref[0, 0].reshape(TN, 4)
    vf = jnp.where(masks_ref[0, 0], 0.0, 1.0).reshape(TN, 1)
    tidx = (jax.lax.broadcasted_iota(jnp.int32, (TN, 1), 0) // N).astype(jnp.float32)

    st1 = jnp.concatenate([bo, tidx], axis=1)
    f1, h1p = _eff(st1, vf)
    b1 = f1[:, :4]
    t1 = f1[:, 4:5]
    dt = jnp.maximum(tidx - t1, 1.0)
    vel = jnp.where(h1p > 0.5, (bo - b1) / dt, 0.0)

    st2 = jnp.concatenate([vel, h1p], axis=1)
    f2, _ = _eff(st2, vf)
    velp = f2[:, :4]
    has2 = jnp.logical_and(f2[:, 4:5] > 0.5, h1p > 0.5)
    acc = jnp.where(has2, (vel - velp) / dt, 0.0)

    mv = jnp.concatenate([bo, vel, acc], axis=1)
    hm = _gelu(jnp.dot(mv, we1_ref[...], preferred_element_type=jnp.float32)
               + bee1_ref[...])
    hm = jnp.dot(hm, we2_ref[...], preferred_element_type=jnp.float32) + bee2_ref[...]

    cnt = vf
    s = 1
    while s < T:
        cnt = cnt + _roll_rows(cnt, s * N)
        s *= 2
    wm = jnp.logical_and(vf > 0.5, cnt >= 2.0)
    mot = jnp.where(wm, hm, 0.0)
    mot_out_ref[0, 0] = mot.reshape(T, N, MD)

    tc = T // ROW_CHUNKS
    rc = tc * N
    for c in range(ROW_CHUNKS):
        x = feat_ref[0, 0, c * tc:(c + 1) * tc].reshape(rc, D)

        y = _ffn(x, wa1_ref, ba1_ref, wa2_ref, ba2_ref)
        x1 = _layernorm(x + y, g1_ref[...], be1_ref[...])

        mc = mot[c * rc:(c + 1) * rc]
        f0 = (jnp.dot(x1.astype(bf), wf1a_ref[...], preferred_element_type=jnp.float32)
              + jnp.dot(mc.astype(bf), wf1b_ref[...], preferred_element_type=jnp.float32)
              + bf1_ref[...])
        f0 = jnp.dot(_gelu(f0.astype(bf)), wf2_ref[...],
                     preferred_element_type=jnp.float32) + bf2_ref[...]

        y2 = _ffn(f0, wn1_ref, bn1_ref, wn2_ref, bn2_ref)
        out = _layernorm(f0 + y2, g2_ref[...], be2_ref[...])
        f_out_ref[0, 0, c * tc:(c + 1) * tc] = out.reshape(tc, N, D)


def kernel(trajectory_features, trajectory_boxes, trajectory_masks, trajectory_times, params):
    p = params
    bf = jnp.bfloat16
    wa1 = p['Wa1'].astype(bf)
    wa2 = p['Wa2'].astype(bf)
    wf1a = p['Wf1'][:D].astype(bf)
    wf1b = p['Wf1'][D:].astype(bf)
    wf2 = p['Wf2'].astype(bf)
    wn1 = p['Wn1'].astype(bf)
    wn2 = p['Wn2'].astype(bf)

    def rep(shape):
        nd = len(shape)
        return pl.BlockSpec(shape, lambda i, _nd=nd: (0,) * _nd)

    in_specs = [
        pl.BlockSpec((1, 1, T, N, D), lambda i: (i // G, i % G, 0, 0, 0)),
        pl.BlockSpec((1, 1, T, N, 4), lambda i: (i // G, i % G, 0, 0, 0)),
        pl.BlockSpec((1, 1, T, N, 1), lambda i: (i // G, i % G, 0, 0, 0)),
        rep((D, D * R)), rep((D * R,)), rep((D * R, D)), rep((D,)),
        rep((D,)), rep((D,)),
        rep((12, MD // 2)), rep((MD // 2,)), rep((MD // 2, MD)), rep((MD,)),
        rep((D, D)), rep((MD, D)), rep((D,)), rep((D, D)), rep((D,)),
        rep((D, D * R)), rep((D * R,)), rep((D * R, D)), rep((D,)),
        rep((D,)), rep((D,)),
    ]
    out_specs = [
        pl.BlockSpec((1, 1, T, N, D), lambda i: (i // G, i % G, 0, 0, 0)),
        pl.BlockSpec((1, 1, T, N, MD), lambda i: (i // G, i % G, 0, 0, 0)),
    ]

    f, mot = pl.pallas_call(
        _fused_kernel,
        grid=(BG,),
        in_specs=in_specs,
        out_specs=out_specs,
        out_shape=[
            jax.ShapeDtypeStruct((B, G, T, N, D), jnp.float32),
            jax.ShapeDtypeStruct((B, G, T, N, MD), jnp.float32),
        ],
    )(trajectory_features, trajectory_boxes, trajectory_masks.reshape(B, G, T, N, 1),
      wa1, p['ba1'], wa2, p['ba2'], p['g1'], p['be1'],
      p['We1'], p['bee1'], p['We2'], p['bee2'],
      wf1a, wf1b, p['bf1'], wf2, p['bf2'],
      wn1, p['bn1'], wn2, p['bn2'], p['g2'], p['be2'])

    return (f, mot)


# speedup vs baseline: 1.0444x; 1.0444x over previous
import jax
import jax.numpy as jnp
from jax.experimental import pallas as pl

B, G, T, N = 4, 8, 32, 100
D = 256
R = 4
MD = 32
TN = T * N
BG = B * G
ROW_CHUNKS = 4


def _shift_rows(x, s):
    return jnp.concatenate([jnp.zeros((s,) + x.shape[1:], x.dtype), x[:-s]], axis=0)


def _roll_rows(x, s):
    return jnp.concatenate([x[-s:], x[:-s]], axis=0)


def _eff(vals, hasf):
    V = vals * hasf
    H = hasf
    s = 1
    while s < T:
        V = jnp.where(H > 0.5, V, _shift_rows(V, s * N))
        H = jnp.maximum(H, _shift_rows(H, s * N))
        s *= 2
    return _shift_rows(V, N), _shift_rows(H, N)


def _gelu(x):
    return x * 0.5 * (1.0 + jax.lax.erf(x * 0.7071067811865476))


def _ffn(x32, w1_ref, b1_ref, w2_ref, b2_ref, chunks=4):
    bf = jnp.bfloat16
    xb = x32.astype(bf)
    hdim = w1_ref.shape[1]
    cs = hdim // chunks
    b1 = b1_ref[...]
    y = jnp.broadcast_to(b2_ref[...], (x32.shape[0], w2_ref.shape[1]))
    for c in range(chunks):
        h = (jnp.dot(xb, w1_ref[:, c * cs:(c + 1) * cs],
                     preferred_element_type=jnp.float32)
             + b1[c * cs:(c + 1) * cs]).astype(bf)
        y = y + jnp.dot(_gelu(h), w2_ref[c * cs:(c + 1) * cs, :],
                        preferred_element_type=jnp.float32)
    return y


def _layernorm(x, g, b):
    m = jnp.mean(x, axis=-1, keepdims=True)
    v = jnp.mean((x - m) ** 2, axis=-1, keepdims=True)
    return (x - m) * jax.lax.rsqrt(v + 1e-5) * g + b


def _fused_kernel(feat_ref, boxes_ref, masks_ref,
                  wa1_ref, ba1_ref, wa2_ref, ba2_ref, g1_ref, be1_ref,
                  we1_ref, bee1_ref, we2_ref, bee2_ref,
                  wf1a_ref, wf1b_ref, bf1_ref, wf2_ref, bf2_ref,
                  wn1_ref, bn1_ref, wn2_ref, bn2_ref, g2_ref, be2_ref,
                  f_out_ref, mot_out_ref):
    bf = jnp.bfloat16

    bo = boxes_ref[0, 0].reshape(TN, 4)
    vf = jnp.where(masks_ref[0, 0], 0.0, 1.0).reshape(TN, 1)
    tidx = (jax.lax.broadcasted_iota(jnp.int32, (TN, 1), 0) // N).astype(jnp.float32)

    st1 = jnp.concatenate([bo, tidx], axis=1)
    f1, h1p = _eff(st1, vf)
    b1 = f1[:, :4]
    t1 = f1[:, 4:5]
    dt = jnp.maximum(tidx - t1, 1.0)
    vel = jnp.where(h1p > 0.5, (bo - b1) / dt, 0.0)

    st2 = jnp.concatenate([vel, h1p], axis=1)
    f2, _ = _eff(st2, vf)
    velp = f2[:, :4]
    has2 = jnp.logical_and(f2[:, 4:5] > 0.5, h1p > 0.5)
    acc = jnp.where(has2, (vel - velp) / dt, 0.0)

    mv = jnp.concatenate([bo, vel, acc], axis=1)
    hm = _gelu(jnp.dot(mv, we1_ref[...], preferred_element_type=jnp.float32)
               + bee1_ref[...])
    hm = jnp.dot(hm, we2_ref[...], preferred_element_type=jnp.float32) + bee2_ref[...]

    cnt = vf
    s = 1
    while s < T:
        cnt = cnt + _roll_rows(cnt, s * N)
        s *= 2
    wm = jnp.logical_and(vf > 0.5, cnt >= 2.0)
    mot = jnp.where(wm, hm, 0.0)
    mot_out_ref[...] = mot.reshape(T, N, MD)

    tc = T // ROW_CHUNKS
    rc = tc * N
    for c in range(ROW_CHUNKS):
        x = feat_ref[c * tc:(c + 1) * tc].reshape(rc, D)

        y = _ffn(x, wa1_ref, ba1_ref, wa2_ref, ba2_ref)
        x1 = _layernorm(x + y, g1_ref[...], be1_ref[...])

        mc = mot[c * rc:(c + 1) * rc]
        f0 = (jnp.dot(x1.astype(bf), wf1a_ref[...], preferred_element_type=jnp.float32)
              + jnp.dot(mc.astype(bf), wf1b_ref[...], preferred_element_type=jnp.float32)
              + bf1_ref[...])
        f0 = jnp.dot(_gelu(f0.astype(bf)), wf2_ref[...],
                     preferred_element_type=jnp.float32) + bf2_ref[...]

        y2 = _ffn(f0, wn1_ref, bn1_ref, wn2_ref, bn2_ref)
        out = _layernorm(f0 + y2, g2_ref[...], be2_ref[...])
        f_out_ref[c * tc:(c + 1) * tc] = out.reshape(tc, N, D)


def kernel(trajectory_features, trajectory_boxes, trajectory_masks, trajectory_times, params):
    p = params
    feat = trajectory_features.reshape(BG * T, N, D)
    bf = jnp.bfloat16
    wa1 = p['Wa1'].astype(bf)
    wa2 = p['Wa2'].astype(bf)
    wf1a = p['Wf1'][:D].astype(bf)
    wf1b = p['Wf1'][D:].astype(bf)
    wf2 = p['Wf2'].astype(bf)
    wn1 = p['Wn1'].astype(bf)
    wn2 = p['Wn2'].astype(bf)

    def rep(shape):
        nd = len(shape)
        return pl.BlockSpec(shape, lambda i, _nd=nd: (0,) * _nd)

    in_specs = [
        pl.BlockSpec((T, N, D), lambda i: (i, 0, 0)),
        pl.BlockSpec((1, 1, T, N, 4), lambda i: (i // G, i % G, 0, 0, 0)),
        pl.BlockSpec((1, 1, T, N, 1), lambda i: (i // G, i % G, 0, 0, 0)),
        rep((D, D * R)), rep((D * R,)), rep((D * R, D)), rep((D,)),
        rep((D,)), rep((D,)),
        rep((12, MD // 2)), rep((MD // 2,)), rep((MD // 2, MD)), rep((MD,)),
        rep((D, D)), rep((MD, D)), rep((D,)), rep((D, D)), rep((D,)),
        rep((D, D * R)), rep((D * R,)), rep((D * R, D)), rep((D,)),
        rep((D,)), rep((D,)),
    ]
    out_specs = [
        pl.BlockSpec((T, N, D), lambda i: (i, 0, 0)),
        pl.BlockSpec((T, N, MD), lambda i: (i, 0, 0)),
    ]

    f, mot = pl.pallas_call(
        _fused_kernel,
        grid=(BG,),
        in_specs=in_specs,
        out_specs=out_specs,
        out_shape=[
            jax.ShapeDtypeStruct((BG * T, N, D), jnp.float32),
            jax.ShapeDtypeStruct((BG * T, N, MD), jnp.float32),
        ],
    )(feat, trajectory_boxes, trajectory_masks.reshape(B, G, T, N, 1),
      wa1, p['ba1'], wa2, p['ba2'], p['g1'], p['be1'],
      p['We1'], p['bee1'], p['We2'], p['bee2'],
      wf1a, wf1b, p['bf1'], wf2, p['bf2'],
      wn1, p['bn1'], wn2, p['bn2'], p['g2'], p['be2'])

    return (f.reshape(B, G, T, N, D), mot.reshape(B, G, T, N, MD))


# speedup vs baseline: 1.0520x; 1.0072x over previous
import jax
import jax.numpy as jnp
from jax.experimental import pallas as pl

B, G, T, N = 4, 8, 32, 100
D = 256
R = 4
MD = 32
TN = T * N
BG = B * G
ROW_CHUNKS = 2


def _shift_rows(x, s):
    return jnp.concatenate([jnp.zeros((s,) + x.shape[1:], x.dtype), x[:-s]], axis=0)


def _roll_rows(x, s):
    return jnp.concatenate([x[-s:], x[:-s]], axis=0)


def _eff(vals, hasf):
    V = vals * hasf
    H = hasf
    s = 1
    while s < T:
        V = jnp.where(H > 0.5, V, _shift_rows(V, s * N))
        H = jnp.maximum(H, _shift_rows(H, s * N))
        s *= 2
    return _shift_rows(V, N), _shift_rows(H, N)


def _gelu(x):
    return x * 0.5 * (1.0 + jax.lax.erf(x * 0.7071067811865476))


def _ffn(x32, w1_ref, b1_ref, w2_ref, b2_ref, chunks=2):
    bf = jnp.bfloat16
    xb = x32.astype(bf)
    hdim = w1_ref.shape[1]
    cs = hdim // chunks
    b1 = b1_ref[...]
    y = jnp.broadcast_to(b2_ref[...], (x32.shape[0], w2_ref.shape[1]))
    for c in range(chunks):
        h = (jnp.dot(xb, w1_ref[:, c * cs:(c + 1) * cs],
                     preferred_element_type=jnp.float32)
             + b1[c * cs:(c + 1) * cs]).astype(bf)
        y = y + jnp.dot(_gelu(h), w2_ref[c * cs:(c + 1) * cs, :],
                        preferred_element_type=jnp.float32)
    return y


def _layernorm(x, g, b):
    m = jnp.mean(x, axis=-1, keepdims=True)
    v = jnp.mean((x - m) ** 2, axis=-1, keepdims=True)
    return (x - m) * jax.lax.rsqrt(v + 1e-5) * g + b


def _fused_kernel(feat_ref, boxes_ref, masks_ref,
                  wa1_ref, ba1_ref, wa2_ref, ba2_ref, g1_ref, be1_ref,
                  we1_ref, bee1_ref, we2_ref, bee2_ref,
                  wf1a_ref, wf1b_ref, bf1_ref, wf2_ref, bf2_ref,
                  wn1_ref, bn1_ref, wn2_ref, bn2_ref, g2_ref, be2_ref,
                  f_out_ref, mot_out_ref):
    bf = jnp.bfloat16

    bo = boxes_ref[0, 0].reshape(TN, 4)
    vf = jnp.where(masks_ref[0, 0], 0.0, 1.0).reshape(TN, 1)
    tidx = (jax.lax.broadcasted_iota(jnp.int32, (TN, 1), 0) // N).astype(jnp.float32)

    st1 = jnp.concatenate([bo, tidx], axis=1)
    f1, h1p = _eff(st1, vf)
    b1 = f1[:, :4]
    t1 = f1[:, 4:5]
    dt = jnp.maximum(tidx - t1, 1.0)
    vel = jnp.where(h1p > 0.5, (bo - b1) / dt, 0.0)

    st2 = jnp.concatenate([vel, h1p], axis=1)
    f2, _ = _eff(st2, vf)
    velp = f2[:, :4]
    has2 = jnp.logical_and(f2[:, 4:5] > 0.5, h1p > 0.5)
    acc = jnp.where(has2, (vel - velp) / dt, 0.0)

    mv = jnp.concatenate([bo, vel, acc], axis=1)
    hm = _gelu(jnp.dot(mv, we1_ref[...], preferred_element_type=jnp.float32)
               + bee1_ref[...])
    hm = jnp.dot(hm, we2_ref[...], preferred_element_type=jnp.float32) + bee2_ref[...]

    cnt = vf
    s = 1
    while s < T:
        cnt = cnt + _roll_rows(cnt, s * N)
        s *= 2
    wm = jnp.logical_and(vf > 0.5, cnt >= 2.0)
    mot = jnp.where(wm, hm, 0.0)
    mot_out_ref[...] = mot.reshape(T, N, MD)

    tc = T // ROW_CHUNKS
    rc = tc * N
    for c in range(ROW_CHUNKS):
        x = feat_ref[c * tc:(c + 1) * tc].reshape(rc, D)

        y = _ffn(x, wa1_ref, ba1_ref, wa2_ref, ba2_ref)
        x1 = _layernorm(x + y, g1_ref[...], be1_ref[...])

        mc = mot[c * rc:(c + 1) * rc]
        f0 = (jnp.dot(x1.astype(bf), wf1a_ref[...], preferred_element_type=jnp.float32)
              + jnp.dot(mc.astype(bf), wf1b_ref[...], preferred_element_type=jnp.float32)
              + bf1_ref[...])
        f0 = jnp.dot(_gelu(f0.astype(bf)), wf2_ref[...],
                     preferred_element_type=jnp.float32) + bf2_ref[...]

        y2 = _ffn(f0, wn1_ref, bn1_ref, wn2_ref, bn2_ref)
        out = _layernorm(f0 + y2, g2_ref[...], be2_ref[...])
        f_out_ref[c * tc:(c + 1) * tc] = out.reshape(tc, N, D)


def kernel(trajectory_features, trajectory_boxes, trajectory_masks, trajectory_times, params):
    p = params
    feat = trajectory_features.reshape(BG * T, N, D)
    bf = jnp.bfloat16
    wa1 = p['Wa1'].astype(bf)
    wa2 = p['Wa2'].astype(bf)
    wf1a = p['Wf1'][:D].astype(bf)
    wf1b = p['Wf1'][D:].astype(bf)
    wf2 = p['Wf2'].astype(bf)
    wn1 = p['Wn1'].astype(bf)
    wn2 = p['Wn2'].astype(bf)

    def rep(shape):
        nd = len(shape)
        return pl.BlockSpec(shape, lambda i, _nd=nd: (0,) * _nd)

    in_specs = [
        pl.BlockSpec((T, N, D), lambda i: (i, 0, 0)),
        pl.BlockSpec((1, 1, T, N, 4), lambda i: (i // G, i % G, 0, 0, 0)),
        pl.BlockSpec((1, 1, T, N, 1), lambda i: (i // G, i % G, 0, 0, 0)),
        rep((D, D * R)), rep((D * R,)), rep((D * R, D)), rep((D,)),
        rep((D,)), rep((D,)),
        rep((12, MD // 2)), rep((MD // 2,)), rep((MD // 2, MD)), rep((MD,)),
        rep((D, D)), rep((MD, D)), rep((D,)), rep((D, D)), rep((D,)),
        rep((D, D * R)), rep((D * R,)), rep((D * R, D)), rep((D,)),
        rep((D,)), rep((D,)),
    ]
    out_specs = [
        pl.BlockSpec((T, N, D), lambda i: (i, 0, 0)),
        pl.BlockSpec((T, N, MD), lambda i: (i, 0, 0)),
    ]

    f, mot = pl.pallas_call(
        _fused_kernel,
        grid=(BG,),
        in_specs=in_specs,
        out_specs=out_specs,
        out_shape=[
            jax.ShapeDtypeStruct((BG * T, N, D), jnp.float32),
            jax.ShapeDtypeStruct((BG * T, N, MD), jnp.float32),
        ],
    )(feat, trajectory_boxes, trajectory_masks.reshape(B, G, T, N, 1),
      wa1, p['ba1'], wa2, p['ba2'], p['g1'], p['be1'],
      p['We1'], p['bee1'], p['We2'], p['bee2'],
      wf1a, wf1b, p['bf1'], wf2, p['bf2'],
      wn1, p['bn1'], wn2, p['bn2'], p['g2'], p['be2'])

    return (f.reshape(B, G, T, N, D), mot.reshape(B, G, T, N, MD))
